# baseline (speedup 1.0000x reference)
import jax
import jax.numpy as jnp
from jax.experimental import pallas as pl
from jax.experimental.pallas import tpu as pltpu

_EMB_D = 768
_BB = 4096
_PR = 2048


def _body(scalars_ref, pool_ref, x_ref, p_hbm, xc_ref, scr_ref, sem):
    i = pl.program_id(0)
    nsteps = pl.num_programs(0)
    B = p_hbm.shape[0]

    @pl.when(i == 0)
    def _fill_and_fire():
        tid = scalars_ref[0]
        valid = scalars_ref[1]
        row = pool_ref[pl.ds(tid, 1), :]
        row = jnp.where(valid == 1, row, jnp.full_like(row, jnp.nan))
        scr_ref[...] = jnp.broadcast_to(row[:, None, :], scr_ref.shape)
        for k in range(B // _PR):
            pltpu.make_async_copy(
                scr_ref, p_hbm.at[pl.ds(k * _PR, _PR)], sem).start()

    xc_ref[...] = x_ref[...]

    @pl.when(i == nsteps - 1)
    def _drain():
        for k in range(B // _PR):
            pltpu.make_async_copy(
                scr_ref, p_hbm.at[pl.ds(k * _PR, _PR)], sem).wait()


def kernel(x_querry, l, x_block, e_p, task_id):
    B = x_querry.shape[0]
    pool = e_p.reshape(e_p.shape[0] * e_p.shape[1], _EMB_D)
    l_i = jnp.asarray(l, jnp.int32)
    valid = ((l_i >= 0) & (l_i < 12)).astype(jnp.int32)
    scalars = jnp.stack([jnp.asarray(task_id, jnp.int32), valid])
    P, xc = pl.pallas_call(
        _body,
        grid_spec=pltpu.PrefetchScalarGridSpec(
            num_scalar_prefetch=1,
            grid=(B // _BB,),
            in_specs=[
                pl.BlockSpec((pool.shape[0], _EMB_D), lambda i, s: (0, 0)),
                pl.BlockSpec((_BB, _EMB_D), lambda i, s: (i, 0)),
            ],
            out_specs=[
                pl.BlockSpec(memory_space=pl.ANY),
                pl.BlockSpec((_BB, _EMB_D), lambda i, s: (i, 0)),
            ],
            scratch_shapes=[
                pltpu.VMEM((_PR, 1, _EMB_D), jnp.float32),
                pltpu.SemaphoreType.DMA,
            ],
        ),
        out_shape=[
            jax.ShapeDtypeStruct((B, 1, _EMB_D), jnp.float32),
            jax.ShapeDtypeStruct((B, _EMB_D), jnp.float32),
        ],
    )(scalars, pool, x_block)
    return (P, xc)
